# baseline (speedup 1.0000x reference)
import functools

import jax
import jax.numpy as jnp
from jax import lax
from jax.experimental import pallas as pl
from jax.experimental.pallas import tpu as pltpu
from jax.experimental.pallas import tpu_sc as plsc

EMBED = 64
B = 16384
NEG = 64

_NC = 2
_NS = 16
_NW = _NC * _NS
_BPW = B // _NW

_G = 8
_BLK = B // _G


def _sc_gather_body(h_hbm, t_hbm, ut_hbm, pt_hbm, pb_hbm, ni_hbm,
                    u_out, t_out, b_out, n_out,
                    idx_h, idx_t, nidx, sem_u, sem_t, sem_b, sem_n):
    c = lax.axis_index("c")
    s = lax.axis_index("s")
    wid = s * _NC + c
    base = wid * _BPW

    pltpu.sync_copy(h_hbm.at[pl.ds(base, _BPW)], idx_h)
    pltpu.sync_copy(t_hbm.at[pl.ds(base, _BPW)], idx_t)

    def step(k, _):
        iv_h = idx_h[pl.ds(k * 16, 16)]
        iv_t = idx_t[pl.ds(k * 16, 16)]
        for lane in range(16):
            j = base + k * 16 + lane
            pltpu.async_copy(
                ut_hbm.at[pl.ds(iv_h[lane], 1), :],
                u_out.at[pl.ds(j, 1), :], sem_u)
            pltpu.async_copy(
                pt_hbm.at[pl.ds(iv_t[lane], 1), :],
                t_out.at[pl.ds(j, 1), :], sem_t)
            pltpu.async_copy(
                pb_hbm.at[pl.ds(iv_t[lane], 1), :],
                b_out.at[pl.ds(j, 1), :], sem_b)
        return ()

    lax.fori_loop(0, _BPW // 16, step, ())

    @pl.when(wid == 0)
    def _():
        pltpu.sync_copy(ni_hbm, nidx)

        def nstep(k, _):
            iv = nidx[pl.ds(k * 16, 16)]
            for lane in range(16):
                pltpu.async_copy(
                    pt_hbm.at[pl.ds(iv[lane], 1), :],
                    n_out.at[pl.ds(k * 16 + lane, 1), :], sem_n)
            return ()

        lax.fori_loop(0, NEG // 16, nstep, ())
        pltpu.make_async_copy(
            pt_hbm.at[pl.ds(0, NEG), :], n_out, sem_n).wait()

    pltpu.make_async_copy(
        ut_hbm.at[pl.ds(0, _BPW), :], u_out.at[pl.ds(base, _BPW)], sem_u).wait()
    pltpu.make_async_copy(
        pt_hbm.at[pl.ds(0, _BPW), :], t_out.at[pl.ds(base, _BPW)], sem_t).wait()
    pltpu.make_async_copy(
        pb_hbm.at[pl.ds(0, _BPW), :], b_out.at[pl.ds(base, _BPW)], sem_b).wait()


def _sc_gather(h_idxs, t_idxs, user_table, product_table, purchase_bias, neg_idxs):
    mesh = plsc.VectorSubcoreMesh(core_axis_name="c", subcore_axis_name="s")
    return pl.kernel(
        _sc_gather_body,
        out_type=(
            jax.ShapeDtypeStruct((B, EMBED), jnp.float32),
            jax.ShapeDtypeStruct((B, EMBED), jnp.float32),
            jax.ShapeDtypeStruct((B, 1), jnp.float32),
            jax.ShapeDtypeStruct((NEG, EMBED), jnp.float32),
        ),
        mesh=mesh,
        scratch_types=[
            pltpu.VMEM((_BPW,), jnp.int32),
            pltpu.VMEM((_BPW,), jnp.int32),
            pltpu.VMEM((NEG,), jnp.int32),
            pltpu.SemaphoreType.DMA,
            pltpu.SemaphoreType.DMA,
            pltpu.SemaphoreType.DMA,
            pltpu.SemaphoreType.DMA,
        ],
    )(h_idxs, t_idxs, user_table, product_table, purchase_bias, neg_idxs)


def _softplus(x):
    return jnp.maximum(x, 0.0) + jnp.log1p(jnp.exp(-jnp.abs(x)))


def _tc_body(u_ref, t_ref, rel_ref, neg_ref, bias_ref, r_ref, out_ref, acc_ref):
    i = pl.program_id(0)

    @pl.when(i == 0)
    def _():
        acc_ref[0] = 0.0
        acc_ref[1] = 0.0

    ex = u_ref[...] + rel_ref[0, :][None, :]
    bias = bias_ref[0, 0, :]
    pos = jnp.sum(ex * t_ref[...], axis=1) + bias
    negl = lax.dot_general(ex, neg_ref[...], (((1,), (1,)), ((), ())),
                           preferred_element_type=jnp.float32) + bias[:, None]
    per_row = _softplus(-pos) + jnp.sum(_softplus(negl), axis=1)
    mask = (r_ref[0, 0, :] == 0).astype(jnp.float32)
    acc_ref[0] += jnp.sum(mask * per_row)
    acc_ref[1] += jnp.sum(mask)

    @pl.when(i == _G - 1)
    def _():
        val = acc_ref[0] / jnp.maximum(acc_ref[1], 1.0) / B
        out_ref[...] = jnp.full((1, 1), val, jnp.float32)


def _tc_compute(u_rows, t_rows, purchase_rel, neg_rows, bias3, r3):
    return pl.pallas_call(
        _tc_body,
        grid=(_G,),
        in_specs=[
            pl.BlockSpec((_BLK, EMBED), lambda i: (i, 0)),
            pl.BlockSpec((_BLK, EMBED), lambda i: (i, 0)),
            pl.BlockSpec((1, EMBED), lambda i: (0, 0)),
            pl.BlockSpec((NEG, EMBED), lambda i: (0, 0)),
            pl.BlockSpec((1, 1, _BLK), lambda i: (i, 0, 0)),
            pl.BlockSpec((1, 1, _BLK), lambda i: (i, 0, 0)),
        ],
        out_specs=pl.BlockSpec((1, 1), lambda i: (0, 0)),
        out_shape=jax.ShapeDtypeStruct((1, 1), jnp.float32),
        scratch_shapes=[pltpu.SMEM((2,), jnp.float32)],
        compiler_params=pltpu.CompilerParams(
            dimension_semantics=("arbitrary",),
        ),
    )(u_rows, t_rows, purchase_rel, neg_rows, bias3, r3)


def kernel(batch_triples, user_table, product_table, purchase_rel, purchase_bias, neg_idxs):
    h_idxs = batch_triples[:, 0]
    r_idxs = batch_triples[:, 1]
    t_idxs = batch_triples[:, 2]

    u_rows, t_rows, bias_g, neg_rows = _sc_gather(
        h_idxs, t_idxs, user_table, product_table, purchase_bias, neg_idxs)

    bias3 = bias_g.reshape(_G, 1, _BLK)
    r3 = r_idxs.reshape(_G, 1, _BLK)
    loss = _tc_compute(u_rows, t_rows, purchase_rel, neg_rows, bias3, r3)
    return loss[0, 0]


# speedup vs baseline: 1.7284x; 1.7284x over previous
import jax
import jax.numpy as jnp
from jax import lax
from jax.experimental import pallas as pl
from jax.experimental.pallas import tpu as pltpu
from jax.experimental.pallas import tpu_sc as plsc

EMBED = 64
B = 16384
NEG = 64

_NC = 2
_NS = 16
_NW = _NC * _NS
_BPW = B // _NW
_CH = 128
_K = _BPW // _CH

_G = 8
_BLK = B // _G


def _sc_user_body(h_hbm, ut_hbm, u_out, idx_h, rows_u, sem_u):
    c = lax.axis_index("c")
    s = lax.axis_index("s")
    wid = s * _NC + c
    base = wid * _BPW

    pltpu.sync_copy(h_hbm.at[pl.ds(wid * _K, _K)], idx_h)
    cps = [
        pltpu.async_copy(
            ut_hbm.at[idx_h.at[j]], rows_u.at[pl.ds(j * _CH, _CH)], sem_u)
        for j in range(_K)
    ]
    for cp in cps:
        cp.wait()
    pltpu.sync_copy(rows_u, u_out.at[pl.ds(base, _BPW)])


def _sc_user_gather(h2, user_table):
    mesh = plsc.VectorSubcoreMesh(core_axis_name="c", subcore_axis_name="s")
    return pl.kernel(
        _sc_user_body,
        out_type=jax.ShapeDtypeStruct((B, EMBED), jnp.float32),
        mesh=mesh,
        scratch_types=[
            pltpu.VMEM((_K, _CH), jnp.int32),
            pltpu.VMEM((_BPW, EMBED), jnp.float32),
            pltpu.SemaphoreType.DMA,
        ],
        compiler_params=pltpu.CompilerParams(use_tc_tiling_on_sc=False),
    )(h2, user_table)


def _sc_prod_body(t_hbm, pt_hbm, pb_hbm, ni_hbm,
                  t_out, b_out, n_out,
                  idx_t, rows_t, bias_v, nidx, nrows,
                  sem_t, sem_b, sem_n):
    c = lax.axis_index("c")
    s = lax.axis_index("s")
    wid = s * _NC + c
    base = wid * _BPW

    pltpu.sync_copy(t_hbm.at[pl.ds(wid * _K, _K)], idx_t)
    cps = []
    for j in range(_K):
        cps.append(pltpu.async_copy(
            pt_hbm.at[idx_t.at[j]], rows_t.at[pl.ds(j * _CH, _CH)], sem_t))
        cps.append(pltpu.async_copy(
            pb_hbm.at[idx_t.at[j]], bias_v.at[pl.ds(j * _CH, _CH)], sem_b))

    @pl.when(wid == 0)
    def _():
        pltpu.sync_copy(ni_hbm, nidx)
        pltpu.async_copy(pt_hbm.at[nidx], nrows, sem_n).wait()
        pltpu.sync_copy(nrows, n_out)

    for cp in cps:
        cp.wait()
    pltpu.sync_copy(rows_t, t_out.at[pl.ds(base, _BPW)])
    pltpu.sync_copy(bias_v, b_out.at[pl.ds(base, _BPW)])


def _sc_prod_gather(t2, product_table, bias1, neg_idxs):
    mesh = plsc.VectorSubcoreMesh(core_axis_name="c", subcore_axis_name="s")
    return pl.kernel(
        _sc_prod_body,
        out_type=(
            jax.ShapeDtypeStruct((B, EMBED), jnp.float32),
            jax.ShapeDtypeStruct((B,), jnp.float32),
            jax.ShapeDtypeStruct((NEG, EMBED), jnp.float32),
        ),
        mesh=mesh,
        scratch_types=[
            pltpu.VMEM((_K, _CH), jnp.int32),
            pltpu.VMEM((_BPW, EMBED), jnp.float32),
            pltpu.VMEM((_BPW,), jnp.float32),
            pltpu.VMEM((NEG,), jnp.int32),
            pltpu.VMEM((NEG, EMBED), jnp.float32),
            pltpu.SemaphoreType.DMA,
            pltpu.SemaphoreType.DMA,
            pltpu.SemaphoreType.DMA,
        ],
        compiler_params=pltpu.CompilerParams(use_tc_tiling_on_sc=False),
    )(t2, product_table, bias1, neg_idxs)


def _softplus(x):
    return jnp.maximum(x, 0.0) + jnp.log1p(jnp.exp(-jnp.abs(x)))


def _tc_body(u_ref, t_ref, rel_ref, neg_ref, bias_ref, r_ref, out_ref, acc_ref):
    i = pl.program_id(0)

    @pl.when(i == 0)
    def _():
        acc_ref[0] = 0.0
        acc_ref[1] = 0.0

    ex = u_ref[...] + rel_ref[0, :][None, :]
    bias = bias_ref[0, 0, :]
    pos = jnp.sum(ex * t_ref[...], axis=1) + bias
    negl = lax.dot_general(ex, neg_ref[...], (((1,), (1,)), ((), ())),
                           preferred_element_type=jnp.float32) + bias[:, None]
    per_row = _softplus(-pos) + jnp.sum(_softplus(negl), axis=1)
    mask = (r_ref[0, 0, :] == 0).astype(jnp.float32)
    acc_ref[0] += jnp.sum(mask * per_row)
    acc_ref[1] += jnp.sum(mask)

    @pl.when(i == _G - 1)
    def _():
        val = acc_ref[0] / jnp.maximum(acc_ref[1], 1.0) / B
        out_ref[...] = jnp.full((1, 1), val, jnp.float32)


def _tc_compute(u_rows, t_rows, purchase_rel, neg_rows, bias3, r3):
    return pl.pallas_call(
        _tc_body,
        grid=(_G,),
        in_specs=[
            pl.BlockSpec((_BLK, EMBED), lambda i: (i, 0)),
            pl.BlockSpec((_BLK, EMBED), lambda i: (i, 0)),
            pl.BlockSpec((1, EMBED), lambda i: (0, 0)),
            pl.BlockSpec((NEG, EMBED), lambda i: (0, 0)),
            pl.BlockSpec((1, 1, _BLK), lambda i: (i, 0, 0)),
            pl.BlockSpec((1, 1, _BLK), lambda i: (i, 0, 0)),
        ],
        out_specs=pl.BlockSpec((1, 1), lambda i: (0, 0)),
        out_shape=jax.ShapeDtypeStruct((1, 1), jnp.float32),
        scratch_shapes=[pltpu.SMEM((2,), jnp.float32)],
        compiler_params=pltpu.CompilerParams(
            dimension_semantics=("arbitrary",),
        ),
    )(u_rows, t_rows, purchase_rel, neg_rows, bias3, r3)


def kernel(batch_triples, user_table, product_table, purchase_rel, purchase_bias, neg_idxs):
    h_idxs = batch_triples[:, 0]
    r_idxs = batch_triples[:, 1]
    t_idxs = batch_triples[:, 2]

    u_rows = _sc_user_gather(h_idxs.reshape(-1, _CH), user_table)
    t_rows, bias_g, neg_rows = _sc_prod_gather(
        t_idxs.reshape(-1, _CH), product_table,
        purchase_bias.reshape(-1), neg_idxs)

    bias3 = bias_g.reshape(_G, 1, _BLK)
    r3 = r_idxs.reshape(_G, 1, _BLK)
    loss = _tc_compute(u_rows, t_rows, purchase_rel, neg_rows, bias3, r3)
    return loss[0, 0]
